# baseline (speedup 1.0000x reference)
import functools

import jax
import jax.numpy as jnp
from jax import lax
from jax.experimental import pallas as pl
from jax.experimental.pallas import tpu as pltpu
from jax.experimental.pallas import tpu_sc as plsc


def _head_body(g_ref, tp_ref, wg1_ref, bg1_ref, wg2_ref, bg2_ref,
               wf1_ref, bf1_ref, wf2_ref, bf2_ref, wo_ref, bo_ref, out_ref):
    g = g_ref[...]
    h = jnp.maximum(jnp.dot(g, wg1_ref[...], preferred_element_type=jnp.float32)
                    + bg1_ref[...], 0.0)
    g2 = jnp.dot(h, wg2_ref[...], preferred_element_type=jnp.float32) + bg2_ref[...]
    xc = jnp.concatenate([g2, tp_ref[...]], axis=1)
    f1 = jnp.maximum(jnp.dot(xc, wf1_ref[...], preferred_element_type=jnp.float32)
                     + bf1_ref[...], 0.0)
    f2 = jnp.maximum(jnp.dot(f1, wf2_ref[...], preferred_element_type=jnp.float32)
                     + bf2_ref[...], 0.0)
    out_ref[...] = jnp.dot(f2, wo_ref[...], preferred_element_type=jnp.float32) + bo_ref[...]


def _mlp_head(g, T, P, Wg1, bg1, Wg2, bg2, Wf1, bf1, Wf2, bf2, Wo, bo):
    B = g.shape[0]
    F = g.shape[1]
    Fp = ((F + 127) // 128) * 128
    g_pad = jnp.pad(g, ((0, 0), (0, Fp - F)))
    wg1 = jnp.pad(Wg1, ((0, Fp - F), (0, 0)))
    tp = jnp.pad(jnp.stack([T, P], axis=1), ((0, 0), (0, 126)))
    wf1 = jnp.pad(Wf1, ((0, 256 - Wf1.shape[0]), (0, 0)))
    wo = jnp.pad(Wo, ((0, 0), (0, 127)))
    bo = jnp.pad(bo, ((0, 127)))
    out = pl.pallas_call(
        _head_body,
        out_shape=jax.ShapeDtypeStruct((B, 128), jnp.float32),
    )(g_pad, tp, wg1, bg1.reshape(1, -1), Wg2, bg2.reshape(1, -1),
      wf1, bf1.reshape(1, -1), Wf2, bf2.reshape(1, -1), wo, bo.reshape(1, -1))
    return out[:, :1]


def _gcn_conv(x, src, dst, dis, W, b):
    n = x.shape[0]
    norm = dis[src] * dis[dst]
    xw = x @ W
    msg = xw[src] * norm[:, None]
    out = jnp.zeros((n, W.shape[1]), xw.dtype).at[dst].add(msg)
    return out + b


def kernel(x, edge_index, batch, T, P, W1, b1, W2, b2, W3, b3,
           Wg1, bg1, Wg2, bg2, Wf1, bf1, Wf2, bf2, Wo, bo):
    n = x.shape[0]
    B = T.shape[0]
    loop = jnp.arange(n, dtype=edge_index.dtype)
    src = jnp.concatenate([edge_index[0], loop])
    dst = jnp.concatenate([edge_index[1], loop])
    deg = jnp.zeros((n,), jnp.float32).at[dst].add(1.0)
    dis = jnp.where(deg > 0, lax.rsqrt(deg), 0.0)
    h = jax.nn.relu(_gcn_conv(x, src, dst, dis, W1, b1))
    h = jax.nn.relu(_gcn_conv(h, src, dst, dis, W2, b2))
    h = jax.nn.relu(_gcn_conv(h, src, dst, dis, W3, b3))
    g = jax.ops.segment_max(h, batch, num_segments=B)
    return _mlp_head(g, T, P, Wg1, bg1, Wg2, bg2, Wf1, bf1, Wf2, bf2, Wo, bo)


# speedup vs baseline: 6.0201x; 6.0201x over previous
import functools

import jax
import jax.numpy as jnp
from jax import lax
from jax.experimental import pallas as pl
from jax.experimental.pallas import tpu as pltpu
from jax.experimental.pallas import tpu_sc as plsc

NC = 2
NS = 16
NP = 50048
E_TOT = 800000
EPT = E_TOT // NS
EW = 2000
NWIN = EPT // EW
RPT = NP // NS


def _scatter_body(nf, y_ref, src_ref, dst_ref, out_ref,
                  idx_v, didx_v, rows_v, acc_sp, sem):
    c = lax.axis_index("c")
    s = lax.axis_index("s")
    ebase = s * EPT

    def chunk_body(jj, _):
        j = c + 2 * jj
        pltpu.sync_copy(y_ref.at[j, pl.ds(s * RPT, RPT)],
                        acc_sp.at[pl.ds(s * RPT, RPT)])
        plsc.subcore_barrier()

        def win_body(w, _):
            base = ebase + w * EW
            pltpu.sync_copy(src_ref.at[pl.ds(base, EW)], idx_v)
            pltpu.sync_copy(dst_ref.at[pl.ds(base, EW)], didx_v)
            pltpu.async_copy(y_ref.at[j].at[idx_v], rows_v, sem).wait()
            pltpu.sync_copy(rows_v, acc_sp.at[didx_v], add=True)
            return 0

        lax.fori_loop(0, NWIN, win_body, 0)
        plsc.subcore_barrier()
        pltpu.sync_copy(acc_sp.at[pl.ds(s * RPT, RPT)],
                        out_ref.at[j, pl.ds(s * RPT, RPT)])
        return 0

    lax.fori_loop(0, (nf - c + 1) // 2, chunk_body, 0)


def _sc_scatter(y_t, src, dst):
    nf = y_t.shape[0]
    mesh = plsc.VectorSubcoreMesh(core_axis_name="c", subcore_axis_name="s")
    return pl.kernel(
        functools.partial(_scatter_body, nf),
        out_type=jax.ShapeDtypeStruct((nf, NP, 16), jnp.float32),
        mesh=mesh,
        compiler_params=pltpu.CompilerParams(use_tc_tiling_on_sc=False),
        scratch_types=[
            pltpu.VMEM((EW,), jnp.int32),
            pltpu.VMEM((EW,), jnp.int32),
            pltpu.VMEM((EW, 16), jnp.float32),
            pltpu.VMEM_SHARED((NP, 16), jnp.float32),
            pltpu.SemaphoreType.DMA,
        ],
    )(y_t, src, dst)


def _gcn_layer(h, src, dst, dis_pad, W, b):
    fout = W.shape[1]
    nf = (fout + 15) // 16
    fp = nf * 16
    y = (h @ W) * dis_pad[:, None]
    y = jnp.pad(y, ((0, 0), (0, fp - fout)))
    y_t = y.reshape(NP, nf, 16).transpose(1, 0, 2)
    s_t = _sc_scatter(y_t, src, dst)
    s = s_t.transpose(1, 0, 2).reshape(NP, fp)[:, :fout]
    return jax.nn.relu(s * dis_pad[:, None] + b)


def _head_body(g_ref, tp_ref, wg1_ref, bg1_ref, wg2_ref, bg2_ref,
               wf1_ref, bf1_ref, wf2_ref, bf2_ref, wo_ref, bo_ref, out_ref):
    g = g_ref[...]
    h = jnp.maximum(jnp.dot(g, wg1_ref[...], preferred_element_type=jnp.float32)
                    + bg1_ref[...], 0.0)
    g2 = jnp.dot(h, wg2_ref[...], preferred_element_type=jnp.float32) + bg2_ref[...]
    xc = jnp.concatenate([g2, tp_ref[...]], axis=1)
    f1 = jnp.maximum(jnp.dot(xc, wf1_ref[...], preferred_element_type=jnp.float32)
                     + bf1_ref[...], 0.0)
    f2 = jnp.maximum(jnp.dot(f1, wf2_ref[...], preferred_element_type=jnp.float32)
                     + bf2_ref[...], 0.0)
    out_ref[...] = jnp.dot(f2, wo_ref[...], preferred_element_type=jnp.float32) + bo_ref[...]


def _mlp_head(g, T, P, Wg1, bg1, Wg2, bg2, Wf1, bf1, Wf2, bf2, Wo, bo):
    B = g.shape[0]
    F = g.shape[1]
    Fp = ((F + 127) // 128) * 128
    g_pad = jnp.pad(g, ((0, 0), (0, Fp - F)))
    wg1 = jnp.pad(Wg1, ((0, Fp - F), (0, 0)))
    tp = jnp.pad(jnp.stack([T, P], axis=1), ((0, 0), (0, 126)))
    wf1 = jnp.pad(Wf1, ((0, 256 - Wf1.shape[0]), (0, 0)))
    wo = jnp.pad(Wo, ((0, 0), (0, 127)))
    bo_p = jnp.pad(bo, ((0, 127)))
    out = pl.pallas_call(
        _head_body,
        out_shape=jax.ShapeDtypeStruct((B, 128), jnp.float32),
    )(g_pad, tp, wg1, bg1.reshape(1, -1), Wg2, bg2.reshape(1, -1),
      wf1, bf1.reshape(1, -1), Wf2, bf2.reshape(1, -1), wo, bo_p.reshape(1, -1))
    return out[:, :1]


def kernel(x, edge_index, batch, T, P, W1, b1, W2, b2, W3, b3,
           Wg1, bg1, Wg2, bg2, Wf1, bf1, Wf2, bf2, Wo, bo):
    n = x.shape[0]
    B = T.shape[0]
    src = edge_index[0]
    dst = edge_index[1]
    deg = jnp.ones((n,), jnp.float32).at[dst].add(1.0)
    dis = lax.rsqrt(deg)
    dis_pad = jnp.pad(dis, (0, NP - n))
    x_pad = jnp.pad(x, ((0, NP - n), (0, 0)))
    h = _gcn_layer(x_pad, src, dst, dis_pad, W1, b1)
    h = _gcn_layer(h, src, dst, dis_pad, W2, b2)
    h = _gcn_layer(h, src, dst, dis_pad, W3, b3)
    g = jax.ops.segment_max(h[:n], batch, num_segments=B)
    return _mlp_head(g, T, P, Wg1, bg1, Wg2, bg2, Wf1, bf1, Wf2, bf2, Wo, bo)


# speedup vs baseline: 9.6188x; 1.5978x over previous
import functools

import jax
import jax.numpy as jnp
from jax import lax
from jax.experimental import pallas as pl
from jax.experimental.pallas import tpu as pltpu
from jax.experimental.pallas import tpu_sc as plsc

NC = 2
NS = 16
NP = 50048
E_TOT = 800000
EPT = E_TOT // NS
EW = 2000
NWIN = EPT // EW
RPT = NP // NS
R = 2176
N_RB = NP // R

ED = E_TOT // (NC * NS)
EWD = 1000
NWD = ED // EWD


def _deg_body(dst_ref, out_ref, ones_v, didx_v, zero_v, acc_sp):
    c = lax.axis_index("c")
    s = lax.axis_index("s")

    def fill(i, _):
        ones_v[pl.ds(i * 16, 16)] = jnp.full((16,), 1.0, jnp.float32)
        return 0
    lax.fori_loop(0, EWD // 16, fill, 0)

    def zfill(i, _):
        zero_v[pl.ds(i * 16, 16)] = jnp.zeros((16,), jnp.float32)
        return 0
    lax.fori_loop(0, RPT // 16, zfill, 0)
    pltpu.sync_copy(zero_v, acc_sp.at[pl.ds(s * RPT, RPT)])
    plsc.subcore_barrier()

    ebase = (c * NS + s) * ED

    def win(w, _):
        pltpu.sync_copy(dst_ref.at[pl.ds(ebase + w * EWD, EWD)], didx_v)
        pltpu.sync_copy(ones_v, acc_sp.at[didx_v], add=True)
        return 0
    lax.fori_loop(0, NWD, win, 0)
    plsc.subcore_barrier()
    pltpu.sync_copy(acc_sp.at[pl.ds(s * RPT, RPT)],
                    out_ref.at[c, pl.ds(s * RPT, RPT)])


def _sc_deg(dst):
    mesh = plsc.VectorSubcoreMesh(core_axis_name="c", subcore_axis_name="s")
    return pl.kernel(
        _deg_body,
        out_type=jax.ShapeDtypeStruct((NC, NP), jnp.float32),
        mesh=mesh,
        compiler_params=pltpu.CompilerParams(use_tc_tiling_on_sc=False),
        scratch_types=[
            pltpu.VMEM((EWD,), jnp.float32),
            pltpu.VMEM((EWD,), jnp.int32),
            pltpu.VMEM((RPT,), jnp.float32),
            pltpu.VMEM_SHARED((NP,), jnp.float32),
        ],
    )(dst)


def _scatter_body(nf, y_ref, src_ref, dst_ref, out_ref,
                  idx_a, didx_a, rows_a, idx_b, didx_b, rows_b,
                  acc_sp, gsem_a, gsem_b):
    c = lax.axis_index("c")
    s = lax.axis_index("s")
    ebase = s * EPT

    def chunk_body(jj, _):
        j = c + 2 * jj
        pltpu.sync_copy(y_ref.at[j, pl.ds(s * RPT, RPT)],
                        acc_sp.at[pl.ds(s * RPT, RPT)])
        plsc.subcore_barrier()

        yj = y_ref.at[j]

        def load(w, idx_v, didx_v):
            base = ebase + w * EW
            pltpu.sync_copy(src_ref.at[pl.ds(base, EW)], idx_v)
            pltpu.sync_copy(dst_ref.at[pl.ds(base, EW)], didx_v)

        load(0, idx_a, didx_a)
        pltpu.async_copy(yj.at[idx_a], rows_a, gsem_a)

        def pair(k, _):
            w = 2 * k
            load(w + 1, idx_b, didx_b)
            pltpu.async_copy(yj.at[idx_b], rows_b, gsem_b)
            pltpu.make_async_copy(yj.at[idx_a], rows_a, gsem_a).wait()
            pltpu.sync_copy(rows_a, acc_sp.at[didx_a], add=True)
            load(w + 2, idx_a, didx_a)
            pltpu.async_copy(yj.at[idx_a], rows_a, gsem_a)
            pltpu.make_async_copy(yj.at[idx_b], rows_b, gsem_b).wait()
            pltpu.sync_copy(rows_b, acc_sp.at[didx_b], add=True)
            return 0

        lax.fori_loop(0, (NWIN - 1) // 2, pair, 0)
        pltpu.make_async_copy(yj.at[idx_a], rows_a, gsem_a).wait()
        pltpu.sync_copy(rows_a, acc_sp.at[didx_a], add=True)

        plsc.subcore_barrier()
        pltpu.sync_copy(acc_sp.at[pl.ds(s * RPT, RPT)],
                        out_ref.at[pl.ds(s * RPT, RPT), pl.ds(j * 16, 16)])
        return 0

    lax.fori_loop(0, (nf - c + 1) // 2, chunk_body, 0)


def _sc_scatter(y_t, src, dst):
    nf = y_t.shape[0]
    mesh = plsc.VectorSubcoreMesh(core_axis_name="c", subcore_axis_name="s")
    return pl.kernel(
        functools.partial(_scatter_body, nf),
        out_type=jax.ShapeDtypeStruct((NP, nf * 16), jnp.float32),
        mesh=mesh,
        compiler_params=pltpu.CompilerParams(use_tc_tiling_on_sc=False),
        scratch_types=[
            pltpu.VMEM((EW,), jnp.int32),
            pltpu.VMEM((EW,), jnp.int32),
            pltpu.VMEM((EW, 16), jnp.float32),
            pltpu.VMEM((EW,), jnp.int32),
            pltpu.VMEM((EW,), jnp.int32),
            pltpu.VMEM((EW, 16), jnp.float32),
            pltpu.VMEM_SHARED((NP, 16), jnp.float32),
            pltpu.SemaphoreType.DMA,
            pltpu.SemaphoreType.DMA,
        ],
    )(y_t, src, dst)


def _dis_body(p_ref, o_ref):
    o_ref[...] = 1.0 / jnp.sqrt(1.0 + p_ref[0] + p_ref[1])


def _tc_dis(partials):
    p = partials.reshape(NC, NP // 128, 128)
    out = pl.pallas_call(
        _dis_body,
        out_shape=jax.ShapeDtypeStruct((NP // 128, 128), jnp.float32),
    )(p)
    return out.reshape(NP, 1)


def _l1_body(x_ref, w_ref, dis_ref, o_ref):
    o_ref[...] = ((jnp.dot(x_ref[...], w_ref[0],
                           preferred_element_type=jnp.float32))
                  * dis_ref[...])[None]


def _tc_layer1(x_pad, W, dis):
    k = x_pad.shape[1]
    nf = W.shape[1] // 16
    w_r = W.reshape(k, nf, 16).transpose(1, 0, 2)
    return pl.pallas_call(
        _l1_body,
        grid=(N_RB, nf),
        in_specs=[
            pl.BlockSpec((R, k), lambda i, j: (i, 0)),
            pl.BlockSpec((1, k, 16), lambda i, j: (j, 0, 0)),
            pl.BlockSpec((R, 1), lambda i, j: (i, 0)),
        ],
        out_specs=pl.BlockSpec((1, R, 16), lambda i, j: (j, i, 0)),
        out_shape=jax.ShapeDtypeStruct((nf, NP, 16), jnp.float32),
    )(x_pad, w_r, dis)


def _mid_body(s_ref, w_ref, dis_ref, b_ref, o_ref, h_scr):
    @pl.when(pl.program_id(1) == 0)
    def _():
        h_scr[...] = jnp.maximum(s_ref[...] * dis_ref[...] + b_ref[...], 0.0)

    o_ref[...] = ((jnp.dot(h_scr[...], w_ref[0],
                           preferred_element_type=jnp.float32))
                  * dis_ref[...])[None]


def _tc_mid(s_prev, W, dis, b_prev):
    fin = s_prev.shape[1]
    nf = W.shape[1] // 16
    w_r = W.reshape(fin, nf, 16).transpose(1, 0, 2)
    return pl.pallas_call(
        _mid_body,
        grid=(N_RB, nf),
        in_specs=[
            pl.BlockSpec((R, fin), lambda i, j: (i, 0)),
            pl.BlockSpec((1, fin, 16), lambda i, j: (j, 0, 0)),
            pl.BlockSpec((R, 1), lambda i, j: (i, 0)),
            pl.BlockSpec((1, fin), lambda i, j: (0, 0)),
        ],
        out_specs=pl.BlockSpec((1, R, 16), lambda i, j: (j, i, 0)),
        out_shape=jax.ShapeDtypeStruct((nf, NP, 16), jnp.float32),
        scratch_shapes=[pltpu.VMEM((R, fin), jnp.float32)],
    )(s_prev, w_r, dis, b_prev)


def _head_body(g_ref, tp_ref, wg1_ref, bg1_ref, wg2_ref, bg2_ref,
               wf1_ref, bf1_ref, wf2_ref, bf2_ref, wo_ref, bo_ref, out_ref):
    g = g_ref[...]
    h = jnp.maximum(jnp.dot(g, wg1_ref[...], preferred_element_type=jnp.float32)
                    + bg1_ref[...], 0.0)
    g2 = jnp.dot(h, wg2_ref[...], preferred_element_type=jnp.float32) + bg2_ref[...]
    xc = jnp.concatenate([g2, tp_ref[...]], axis=1)
    f1 = jnp.maximum(jnp.dot(xc, wf1_ref[...], preferred_element_type=jnp.float32)
                     + bf1_ref[...], 0.0)
    f2 = jnp.maximum(jnp.dot(f1, wf2_ref[...], preferred_element_type=jnp.float32)
                     + bf2_ref[...], 0.0)
    out_ref[...] = jnp.dot(f2, wo_ref[...], preferred_element_type=jnp.float32) + bo_ref[...]


def _mlp_head(g, T, P, Wg1, bg1, Wg2, bg2, Wf1, bf1, Wf2, bf2, Wo, bo):
    B = g.shape[0]
    F = g.shape[1]
    Fp = ((F + 127) // 128) * 128
    g_pad = jnp.pad(g, ((0, 0), (0, Fp - F)))
    wg1 = jnp.pad(Wg1, ((0, Fp - F), (0, 0)))
    tp = jnp.pad(jnp.stack([T, P], axis=1), ((0, 0), (0, 126)))
    wf1 = jnp.pad(Wf1, ((0, 256 - Wf1.shape[0]), (0, 0)))
    wo = jnp.pad(Wo, ((0, 0), (0, 127)))
    bo_p = jnp.pad(bo, ((0, 127)))
    out = pl.pallas_call(
        _head_body,
        out_shape=jax.ShapeDtypeStruct((B, 128), jnp.float32),
    )(g_pad, tp, wg1, bg1.reshape(1, -1), Wg2, bg2.reshape(1, -1),
      wf1, bf1.reshape(1, -1), Wf2, bf2.reshape(1, -1), wo, bo_p.reshape(1, -1))
    return out[:, :1]


def _pad_w(W, b):
    fin, fout = W.shape
    fi = ((fin + 15) // 16) * 16
    fo = ((fout + 15) // 16) * 16
    return (jnp.pad(W, ((0, fi - fin), (0, fo - fout))),
            jnp.pad(b, (0, fo - fout)).reshape(1, fo))


def kernel(x, edge_index, batch, T, P, W1, b1, W2, b2, W3, b3,
           Wg1, bg1, Wg2, bg2, Wf1, bf1, Wf2, bf2, Wo, bo):
    n = x.shape[0]
    B = T.shape[0]
    src = edge_index[0]
    dst = edge_index[1]

    partials = _sc_deg(dst)
    dis = _tc_dis(partials)

    x_pad = jnp.pad(x, ((0, NP - n), (0, 128 - x.shape[1])))
    w1, b1p = _pad_w(W1, b1)
    w1 = jnp.pad(w1, ((0, 128 - w1.shape[0]), (0, 0)))
    w2, b2p = _pad_w(W2, b2)
    w3, b3p = _pad_w(W3, b3)

    y1_t = _tc_layer1(x_pad, w1, dis)
    s1 = _sc_scatter(y1_t, src, dst)
    y2_t = _tc_mid(s1, w2, dis, b1p)
    s2 = _sc_scatter(y2_t, src, dst)
    y3_t = _tc_mid(s2, w3, dis, b2p)
    s3 = _sc_scatter(y3_t, src, dst)

    h3 = jnp.maximum(s3 * dis + b3p, 0.0)
    g = jax.ops.segment_max(h3[:n, :W3.shape[1]], batch, num_segments=B)
    return _mlp_head(g, T, P, Wg1, bg1, Wg2, bg2, Wf1, bf1, Wf2, bf2, Wo, bo)
